# baseline (speedup 1.0000x reference)
import jax
import jax.numpy as jnp
import numpy as np
from jax import lax
from jax.experimental import pallas as pl
from jax.experimental.pallas import tpu as pltpu
from jax.experimental.pallas import tpu_sc as plsc

VOCAB = 100000
DIM = 768
MAXLEN = 2048
NTYPES = 2
FREQ = 10000.0
B = 4
S = 2048
PAD_POS = 0
EPS = 1e-12

N = B * S
NC, NS = 2, 16
NW = NC * NS
CHUNK = 32
LANES = 16
DVEC = DIM // LANES


def _pe_table():
    pos = np.arange(MAXLEN, dtype=np.float32)[:, None]
    i = np.arange(DIM // 2, dtype=np.float32)
    div = np.power(FREQ, 2.0 * i / DIM, dtype=np.float32)
    ang = (pos / div).astype(np.float32)
    pe = np.zeros((MAXLEN, DIM), dtype=np.float32)
    pe[:, 0::2] = np.sin(ang)
    pe[:, 1::2] = np.cos(ang)
    pe[PAD_POS] = 0.0
    return jnp.asarray(pe)


def _make_sc_body(nrows):
    tok_per_w = nrows // NW
    nchunk = tok_per_w // CHUNK

    def body(tok_table, pe_table, tok_ids, pos_ids, emb_out,
             itok0, itok1, ipos0, ipos1, a0, a1, b0, b1,
             sg0, sg1, ss0, ss1):
        itok = (itok0, itok1)
        ipos = (ipos0, ipos1)
        abuf = (a0, a1)
        bbuf = (b0, b1)
        sem_g = (sg0, sg1)
        sem_s = (ss0, ss1)

        wid = lax.axis_index("s") * NC + lax.axis_index("c")
        w0 = wid * tok_per_w

        def stage_idx(c, p):
            base = w0 + c * CHUNK
            pltpu.sync_copy(tok_ids.at[pl.ds(base, CHUNK)], itok[p])
            pltpu.sync_copy(pos_ids.at[pl.ds(base, CHUNK)], ipos[p])

        def start_gather(p):
            pltpu.async_copy(tok_table.at[itok[p]], abuf[p], sem_g[p])
            pltpu.async_copy(pe_table.at[ipos[p]], bbuf[p], sem_g[p])

        def wait_gather(p):
            pltpu.make_async_copy(tok_table.at[itok[p]], abuf[p], sem_g[p]).wait()
            pltpu.make_async_copy(pe_table.at[ipos[p]], bbuf[p], sem_g[p]).wait()

        def start_store(c, p):
            base = w0 + c * CHUNK
            pltpu.async_copy(abuf[p], emb_out.at[pl.ds(base, CHUNK)], sem_s[p])

        def wait_store(c, p):
            base = w0 + c * CHUNK
            pltpu.make_async_copy(
                abuf[p], emb_out.at[pl.ds(base, CHUNK)], sem_s[p]).wait()

        def add_chunk(p):
            A = abuf[p]
            Bb = bbuf[p]

            @plsc.parallel_loop(0, CHUNK)
            def row(r):
                for j in range(DVEC):
                    sl = pl.ds(j * LANES, LANES)
                    plsc.addupdate(A.at[r, sl], Bb[r, sl])

        stage_idx(0, 0)
        start_gather(0)
        stage_idx(1, 1)
        start_gather(1)
        wait_gather(0)
        add_chunk(0)
        start_store(0, 0)

        def steady(cc, _):
            for poff in range(2):
                c = 1 + cc * 2 + poff
                p = (1 + poff) % 2
                np_ = 1 - p
                wait_gather(p)
                wait_store(c - 1, np_)
                stage_idx(c + 1, np_)
                start_gather(np_)
                add_chunk(p)
                start_store(c, p)
            return 0

        lax.fori_loop(0, (nchunk - 2) // 2, steady, 0)

        pl_last = (nchunk - 1) % 2
        wait_gather(pl_last)
        add_chunk(pl_last)
        start_store(nchunk - 1, pl_last)
        wait_store(nchunk - 2, 1 - pl_last)
        wait_store(nchunk - 1, pl_last)

    return body


def _sc_gather_sum(tok_table, pe_table, tok_ids, pos_ids):
    nrows = tok_ids.shape[0]
    mesh = plsc.VectorSubcoreMesh(
        core_axis_name="c", subcore_axis_name="s",
        num_cores=NC, num_subcores=NS)
    f = pl.kernel(
        _make_sc_body(nrows),
        out_type=jax.ShapeDtypeStruct((nrows, DIM), jnp.float32),
        mesh=mesh,
        compiler_params=pltpu.CompilerParams(needs_layout_passes=False),
        scratch_types=[
            pltpu.VMEM((CHUNK,), jnp.int32),
            pltpu.VMEM((CHUNK,), jnp.int32),
            pltpu.VMEM((CHUNK,), jnp.int32),
            pltpu.VMEM((CHUNK,), jnp.int32),
            pltpu.VMEM((CHUNK, DIM), jnp.float32),
            pltpu.VMEM((CHUNK, DIM), jnp.float32),
            pltpu.VMEM((CHUNK, DIM), jnp.float32),
            pltpu.VMEM((CHUNK, DIM), jnp.float32),
            pltpu.SemaphoreType.DMA,
            pltpu.SemaphoreType.DMA,
            pltpu.SemaphoreType.DMA,
            pltpu.SemaphoreType.DMA,
        ],
    )
    return f(tok_table, pe_table, tok_ids, pos_ids)


ROWS_BLK = 512


def _ln_body(emb_ref, tt_ref, seg_ref, gamma_ref, beta_ref, out_ref):
    t = tt_ref[...]
    seg0 = seg_ref[0:1, :]
    seg1 = seg_ref[1:2, :]
    x = emb_ref[...] + seg0 + t * (seg1 - seg0)
    mu = jnp.mean(x, axis=-1, keepdims=True)
    xc = x - mu
    var = jnp.mean(xc * xc, axis=-1, keepdims=True)
    inv = 1.0 / jnp.sqrt(var + EPS)
    out_ref[0] = xc * inv * gamma_ref[...][None, :] + beta_ref[...][None, :]


def _layernorm(emb, tt_f32, seg_table, gamma, beta):
    blk_per_s = S // ROWS_BLK
    grid = (N // ROWS_BLK,)
    return pl.pallas_call(
        _ln_body,
        grid=grid,
        in_specs=[
            pl.BlockSpec((ROWS_BLK, DIM), lambda i: (i, 0)),
            pl.BlockSpec((ROWS_BLK, 1), lambda i: (i, 0)),
            pl.BlockSpec((NTYPES, DIM), lambda i: (0, 0)),
            pl.BlockSpec((DIM,), lambda i: (0,)),
            pl.BlockSpec((DIM,), lambda i: (0,)),
        ],
        out_specs=pl.BlockSpec(
            (1, ROWS_BLK, DIM),
            lambda i: (i // blk_per_s, i % blk_per_s, 0)),
        out_shape=jax.ShapeDtypeStruct((B, S, DIM), jnp.float32),
    )(emb, tt_f32, seg_table, gamma, beta)


def kernel(input_ids, position_ids, token_type_ids, token_table,
           segment_table, gamma, beta):
    pe_table = _pe_table()
    tok_ids = input_ids.reshape(N).astype(jnp.int32)
    pos_ids = position_ids.reshape(N).astype(jnp.int32)
    tt_f32 = token_type_ids.reshape(N, 1).astype(jnp.float32)
    emb = _sc_gather_sum(token_table, pe_table, tok_ids, pos_ids)
    return _layernorm(emb, tt_f32, segment_table, gamma, beta)


# speedup vs baseline: 1.0561x; 1.0561x over previous
import jax
import jax.numpy as jnp
import numpy as np
from jax import lax
from jax.experimental import pallas as pl
from jax.experimental.pallas import tpu as pltpu
from jax.experimental.pallas import tpu_sc as plsc

VOCAB = 100000
DIM = 768
MAXLEN = 2048
NTYPES = 2
FREQ = 10000.0
B = 4
S = 2048
PAD_POS = 0
EPS = 1e-12

N = B * S
NC, NS = 2, 16
NW = NC * NS
CHUNK = 32
LANES = 16
DVEC = DIM // LANES


def _pe_table():
    pos = np.arange(MAXLEN, dtype=np.float32)[:, None]
    i = np.arange(DIM // 2, dtype=np.float32)
    div = np.power(FREQ, 2.0 * i / DIM, dtype=np.float32)
    ang = (pos / div).astype(np.float32)
    pe = np.zeros((MAXLEN, DIM), dtype=np.float32)
    pe[:, 0::2] = np.sin(ang)
    pe[:, 1::2] = np.cos(ang)
    pe[PAD_POS] = 0.0
    return jnp.asarray(pe)


def _make_sc_body(nrows):
    tok_per_w = nrows // NW
    nchunk = tok_per_w // CHUNK

    def body(tok_table, pe_table, tok_ids, pos_ids, emb_out,
             itok0, itok1, ipos0, ipos1, a0, a1, b0, b1,
             sg0, sg1, ss0, ss1):
        itok = (itok0, itok1)
        ipos = (ipos0, ipos1)
        abuf = (a0, a1)
        bbuf = (b0, b1)
        sem_g = (sg0, sg1)
        sem_s = (ss0, ss1)

        wid = lax.axis_index("s") * NC + lax.axis_index("c")
        w0 = wid * tok_per_w

        def stage_idx(c, p):
            base = w0 + c * CHUNK
            pltpu.sync_copy(tok_ids.at[pl.ds(base, CHUNK)], itok[p])
            pltpu.sync_copy(pos_ids.at[pl.ds(base, CHUNK)], ipos[p])

        def start_gather(p):
            pltpu.async_copy(tok_table.at[itok[p]], abuf[p], sem_g[p])
            pltpu.async_copy(pe_table.at[ipos[p]], bbuf[p], sem_g[p])

        def wait_gather(p):
            pltpu.make_async_copy(tok_table.at[itok[p]], abuf[p], sem_g[p]).wait()
            pltpu.make_async_copy(pe_table.at[ipos[p]], bbuf[p], sem_g[p]).wait()

        def start_store(c, p):
            base = w0 + c * CHUNK
            pltpu.async_copy(abuf[p], emb_out.at[pl.ds(base, CHUNK)], sem_s[p])

        def wait_store(c, p):
            base = w0 + c * CHUNK
            pltpu.make_async_copy(
                abuf[p], emb_out.at[pl.ds(base, CHUNK)], sem_s[p]).wait()

        def add_chunk(p):
            A = abuf[p]
            Bb = bbuf[p]

            @plsc.parallel_loop(0, CHUNK)
            def row(r):
                for j in range(DVEC):
                    sl = pl.ds(j * LANES, LANES)
                    plsc.addupdate(A.at[r, sl], Bb[r, sl])

        stage_idx(0, 0)
        start_gather(0)
        stage_idx(1, 1)
        start_gather(1)
        wait_gather(0)
        add_chunk(0)
        start_store(0, 0)

        def steady(cc, _):
            for poff in range(2):
                c = 1 + cc * 2 + poff
                p = (1 + poff) % 2
                np_ = 1 - p
                wait_gather(p)
                wait_store(c - 1, np_)
                stage_idx(c + 1, np_)
                start_gather(np_)
                add_chunk(p)
                start_store(c, p)
            return 0

        lax.fori_loop(0, (nchunk - 2) // 2, steady, 0)

        pl_last = (nchunk - 1) % 2
        wait_gather(pl_last)
        add_chunk(pl_last)
        start_store(nchunk - 1, pl_last)
        wait_store(nchunk - 2, 1 - pl_last)
        wait_store(nchunk - 1, pl_last)

    return body


def _sc_gather_sum(tok_table, pe_table, tok_ids, pos_ids):
    nrows = tok_ids.shape[0]
    mesh = plsc.VectorSubcoreMesh(
        core_axis_name="c", subcore_axis_name="s",
        num_cores=NC, num_subcores=NS)
    f = pl.kernel(
        _make_sc_body(nrows),
        out_type=jax.ShapeDtypeStruct((nrows, DIM), jnp.float32),
        mesh=mesh,
        compiler_params=pltpu.CompilerParams(needs_layout_passes=False),
        scratch_types=[
            pltpu.VMEM((CHUNK,), jnp.int32),
            pltpu.VMEM((CHUNK,), jnp.int32),
            pltpu.VMEM((CHUNK,), jnp.int32),
            pltpu.VMEM((CHUNK,), jnp.int32),
            pltpu.VMEM((CHUNK, DIM), jnp.float32),
            pltpu.VMEM((CHUNK, DIM), jnp.float32),
            pltpu.VMEM((CHUNK, DIM), jnp.float32),
            pltpu.VMEM((CHUNK, DIM), jnp.float32),
            pltpu.SemaphoreType.DMA,
            pltpu.SemaphoreType.DMA,
            pltpu.SemaphoreType.DMA,
            pltpu.SemaphoreType.DMA,
        ],
    )
    return f(tok_table, pe_table, tok_ids, pos_ids)


ROWS_BLK = 2048


def _ln_body(emb_ref, tt_ref, seg_ref, gamma_ref, beta_ref, out_ref):
    t = tt_ref[...]
    seg0 = seg_ref[0:1, :]
    seg1 = seg_ref[1:2, :]
    x = emb_ref[...] + seg0 + t * (seg1 - seg0)
    mu = jnp.mean(x, axis=-1, keepdims=True)
    xc = x - mu
    var = jnp.mean(xc * xc, axis=-1, keepdims=True)
    inv = 1.0 / jnp.sqrt(var + EPS)
    y = xc * inv * gamma_ref[...][None, :] + beta_ref[...][None, :]
    out_ref[...] = y.reshape(out_ref.shape)


def _layernorm(emb, tt_f32, seg_table, gamma, beta):
    grid = (N // ROWS_BLK,)
    return pl.pallas_call(
        _ln_body,
        grid=grid,
        in_specs=[
            pl.BlockSpec((ROWS_BLK, DIM), lambda i: (i, 0)),
            pl.BlockSpec((ROWS_BLK, 1), lambda i: (i, 0)),
            pl.BlockSpec((NTYPES, DIM), lambda i: (0, 0)),
            pl.BlockSpec((DIM,), lambda i: (0,)),
            pl.BlockSpec((DIM,), lambda i: (0,)),
        ],
        out_specs=pl.BlockSpec(
            (ROWS_BLK // S, S, DIM), lambda i: (i, 0, 0)),
        out_shape=jax.ShapeDtypeStruct((B, S, DIM), jnp.float32),
    )(emb, tt_f32, seg_table, gamma, beta)


def kernel(input_ids, position_ids, token_type_ids, token_table,
           segment_table, gamma, beta):
    pe_table = _pe_table()
    tok_ids = input_ids.reshape(N).astype(jnp.int32)
    pos_ids = position_ids.reshape(N).astype(jnp.int32)
    tt_f32 = token_type_ids.reshape(N, 1).astype(jnp.float32)
    emb = _sc_gather_sum(token_table, pe_table, tok_ids, pos_ids)
    return _layernorm(emb, tt_f32, segment_table, gamma, beta)
